# baseline (speedup 1.0000x reference)
import functools

import jax
import jax.numpy as jnp
from jax import lax
from jax.experimental import pallas as pl
from jax.experimental.pallas import tpu as pltpu
from jax.experimental.pallas import tpu_sc as plsc

B = 1024
T = 50
P = 20
ED = 16
TD = 16

NC = 2
NS = 16
NW = NC * NS

N_TEXT = B * T
N_EV = B * P + B
TEXT_PER_W = N_TEXT // NW
EV_PER_W = N_EV // NW
GCHUNK = 128


def _chunks(total):
    offs = []
    o = 0
    while o < total:
        offs.append((o, min(GCHUNK, total - o)))
        o += GCHUNK
    return offs


def _sc_gather(text_table, event_table, text_idx, ev_idx):
    mesh = plsc.VectorSubcoreMesh(core_axis_name="c", subcore_axis_name="s")

    @functools.partial(
        pl.kernel,
        mesh=mesh,
        compiler_params=pltpu.CompilerParams(use_tc_tiling_on_sc=False),
        out_type=(
            jax.ShapeDtypeStruct((N_TEXT, TD), jnp.float32),
            jax.ShapeDtypeStruct((N_EV, ED), jnp.float32),
        ),
        scratch_types=[
            pltpu.VMEM((TEXT_PER_W,), jnp.int32),
            pltpu.VMEM((EV_PER_W,), jnp.int32),
            pltpu.VMEM((TEXT_PER_W, TD), jnp.float32),
            pltpu.VMEM((EV_PER_W, ED), jnp.float32),
            pltpu.SemaphoreType.DMA,
        ],
    )
    def k(tt_hbm, et_hbm, ti_hbm, ei_hbm, out_t_hbm, out_e_hbm,
          ti_v, ei_v, rt_v, re_v, sem):
        wid = lax.axis_index("s") * NC + lax.axis_index("c")
        tbase = wid * TEXT_PER_W
        ebase = wid * EV_PER_W
        pltpu.sync_copy(ti_hbm.at[pl.ds(tbase, TEXT_PER_W)], ti_v)
        pltpu.sync_copy(ei_hbm.at[pl.ds(ebase, EV_PER_W)], ei_v)
        handles = []
        for off, sz in _chunks(TEXT_PER_W):
            handles.append(pltpu.async_copy(
                tt_hbm.at[ti_v.at[pl.ds(off, sz)]],
                rt_v.at[pl.ds(off, sz)], sem))
        for off, sz in _chunks(EV_PER_W):
            handles.append(pltpu.async_copy(
                et_hbm.at[ei_v.at[pl.ds(off, sz)]],
                re_v.at[pl.ds(off, sz)], sem))
        for h in handles:
            h.wait()
        pltpu.sync_copy(rt_v, out_t_hbm.at[pl.ds(tbase, TEXT_PER_W)])
        pltpu.sync_copy(re_v, out_e_hbm.at[pl.ds(ebase, EV_PER_W)])

    return k(text_table, event_table, text_idx, ev_idx)


def _pool_body(gt_ref, gp_ref, ge_ref, tl_ref, pl_ref, out_ref):
    gt = gt_ref[...]
    tl = tl_ref[...]
    col = lax.broadcasted_iota(jnp.int32, (B, T * TD), 1)
    mask = (lax.shift_right_logical(col, 4) < tl).astype(jnp.float32)
    sr = lax.broadcasted_iota(jnp.int32, (T * TD, TD), 0)
    sc = lax.broadcasted_iota(jnp.int32, (T * TD, TD), 1)
    sel_t = ((sr & (TD - 1)) == sc).astype(jnp.float32)
    pooled_t = lax.dot_general(
        gt * mask, sel_t, (((1,), (0,)), ((), ())),
        preferred_element_type=jnp.float32, precision=lax.Precision.HIGHEST)
    enc_t = pooled_t / jnp.maximum(tl.astype(jnp.float32), 1.0)

    gp = gp_ref[...]
    pl_ = pl_ref[...]
    colp = lax.broadcasted_iota(jnp.int32, (B, P * ED), 1)
    maskp = (lax.shift_right_logical(colp, 4) < pl_).astype(jnp.float32)
    pr = lax.broadcasted_iota(jnp.int32, (P * ED, ED), 0)
    pc = lax.broadcasted_iota(jnp.int32, (P * ED, ED), 1)
    sel_p = ((pr & (ED - 1)) == pc).astype(jnp.float32)
    pooled_p = lax.dot_general(
        gp * maskp, sel_p, (((1,), (0,)), ((), ())),
        preferred_element_type=jnp.float32, precision=lax.Precision.HIGHEST)
    enc_p = pooled_p / jnp.maximum(pl_.astype(jnp.float32), 1.0)

    out_ref[...] = jnp.concatenate([ge_ref[...], enc_t, enc_p], axis=1)


def _matmul_body(mlp_ref, w_ref, b_ref, out_ref):
    acc = lax.dot_general(
        mlp_ref[...], w_ref[...], (((1,), (1,)), ((), ())),
        preferred_element_type=jnp.float32, precision=lax.Precision.HIGHEST)
    out_ref[...] = acc + b_ref[...]


BN = 2048


def kernel(e1, e1_text_tokens, e1_text_lengths, e1prev_tokens, e1prev_lengths,
           event_table, text_table, W, b):
    EV = W.shape[0]
    text_idx = e1_text_tokens.reshape(-1).astype(jnp.int32)
    ev_idx = jnp.concatenate(
        [e1prev_tokens.reshape(-1), e1]).astype(jnp.int32)

    rows_t, rows_e = _sc_gather(text_table, event_table, text_idx, ev_idx)

    gt = rows_t.reshape(B, T * TD)
    gp = rows_e[:B * P].reshape(B, P * ED)
    ge = rows_e[B * P:]

    mlp = pl.pallas_call(
        _pool_body,
        out_shape=jax.ShapeDtypeStruct((B, ED + TD + ED), jnp.float32),
    )(gt, gp, ge,
      e1_text_lengths.reshape(B, 1).astype(jnp.int32),
      e1prev_lengths.reshape(B, 1).astype(jnp.int32))

    nblk = (EV + BN - 1) // BN
    logits = pl.pallas_call(
        _matmul_body,
        grid=(nblk,),
        in_specs=[
            pl.BlockSpec((B, ED + TD + ED), lambda i: (0, 0)),
            pl.BlockSpec((BN, ED + TD + ED), lambda i: (i, 0)),
            pl.BlockSpec((1, BN), lambda i: (0, i)),
        ],
        out_specs=pl.BlockSpec((B, BN), lambda i: (0, i)),
        out_shape=jax.ShapeDtypeStruct((B, EV), jnp.float32),
        compiler_params=pltpu.CompilerParams(
            dimension_semantics=("arbitrary",)),
    )(mlp, W, b.reshape(1, EV))
    return logits


# speedup vs baseline: 1.4589x; 1.4589x over previous
import dataclasses
import functools

import jax
import jax.numpy as jnp
from jax import lax
from jax.experimental import pallas as pl
from jax.experimental.pallas import tpu as pltpu
from jax.experimental.pallas import tpu_sc as plsc

B = 1024
T = 50
P = 20
ED = 16
TD = 16
FD = ED + TD + ED

NC = 2
NS = 16
NW = NC * NS
BW = B // NW
TEXT_PER_W = BW * T
PREV_PER_W = BW * P
GCHUNK = 128
PAD_ROW = 1


def _chunks(total):
    offs = []
    o = 0
    while o < total:
        offs.append((o, min(GCHUNK, total - o)))
        o += GCHUNK
    return offs


def _iota16():
    return lax.broadcasted_iota(jnp.int32, (16,), 0)


def _mask_indices(idx_v, len_v, n, stride):
    iota = _iota16()
    pad = jnp.full((16,), PAD_ROW, jnp.int32)
    stride_v = jnp.full((16,), stride, jnp.int32)

    @pl.loop(0, n // 16)
    def _(k):
        p = iota + k * 16
        belem = lax.div(p, stride_v)
        pos = p - belem * stride_v
        lens = plsc.load_gather(len_v, [belem])
        idx16 = idx_v[pl.ds(k * 16, 16)]
        idx_v[pl.ds(k * 16, 16)] = jnp.where(pos < lens, idx16, pad)


def _sc_compiler_params():
    cp = pltpu.CompilerParams(use_tc_tiling_on_sc=False)
    if "needs_layout_passes" in pltpu.CompilerParams.__dataclass_fields__:
        cp = dataclasses.replace(cp, needs_layout_passes=False)
    return cp


def _sc_gather_pool(text_table, event_table, text_idx, prev_idx, e1,
                    tlen, plen):
    mesh = plsc.VectorSubcoreMesh(core_axis_name="c", subcore_axis_name="s")

    @functools.partial(
        pl.kernel,
        mesh=mesh,
        compiler_params=_sc_compiler_params(),
        out_type=jax.ShapeDtypeStruct((B, FD), jnp.float32),
        scratch_types=[
            pltpu.VMEM((TEXT_PER_W,), jnp.int32),
            pltpu.VMEM((PREV_PER_W,), jnp.int32),
            pltpu.VMEM((BW,), jnp.int32),
            pltpu.VMEM((BW,), jnp.int32),
            pltpu.VMEM((BW,), jnp.int32),
            pltpu.VMEM((TEXT_PER_W, TD), jnp.float32),
            pltpu.VMEM((PREV_PER_W, ED), jnp.float32),
            pltpu.VMEM((BW, ED), jnp.float32),
            pltpu.VMEM((BW, FD), jnp.float32),
            pltpu.SemaphoreType.DMA,
        ],
    )
    def k(tt_hbm, et_hbm, ti_hbm, pi_hbm, e1_hbm, tl_hbm, pl_hbm, out_hbm,
          ti_v, pi_v, e1_v, tl_v, pl_v, rt_v, rp_v, re_v, mlp_v, sem):
        wid = lax.axis_index("s") * NC + lax.axis_index("c")
        b0 = wid * BW
        pltpu.sync_copy(ti_hbm.at[pl.ds(b0 * T, TEXT_PER_W)], ti_v)
        pltpu.sync_copy(pi_hbm.at[pl.ds(b0 * P, PREV_PER_W)], pi_v)
        pltpu.sync_copy(e1_hbm.at[pl.ds(b0, BW)], e1_v)
        pltpu.sync_copy(tl_hbm.at[pl.ds(b0, BW)], tl_v)
        pltpu.sync_copy(pl_hbm.at[pl.ds(b0, BW)], pl_v)

        _mask_indices(ti_v, tl_v, TEXT_PER_W, T)
        _mask_indices(pi_v, pl_v, PREV_PER_W, P)

        handles = []
        for off, sz in _chunks(TEXT_PER_W):
            handles.append(pltpu.async_copy(
                tt_hbm.at[ti_v.at[pl.ds(off, sz)]],
                rt_v.at[pl.ds(off, sz)], sem))
        for off, sz in _chunks(PREV_PER_W):
            handles.append(pltpu.async_copy(
                et_hbm.at[pi_v.at[pl.ds(off, sz)]],
                rp_v.at[pl.ds(off, sz)], sem))
        handles.append(pltpu.async_copy(et_hbm.at[e1_v], re_v, sem))
        for h in handles:
            h.wait()

        one = jnp.full((16,), 1.0, jnp.float32)

        @pl.loop(0, BW)
        def _(j):
            mlp_v[j, pl.ds(0, ED)] = re_v[j, :]

            acc = jnp.zeros((TD,), jnp.float32)
            tb = j * T
            for t in range(T):
                acc = acc + rt_v[tb + t, :]
            ln = plsc.load_gather(tl_v, [jnp.full((16,), j, jnp.int32)])
            den = jnp.maximum(ln.astype(jnp.float32), one)
            mlp_v[j, pl.ds(ED, TD)] = acc / den

            acc2 = jnp.zeros((ED,), jnp.float32)
            pb = j * P
            for t in range(P):
                acc2 = acc2 + rp_v[pb + t, :]
            ln2 = plsc.load_gather(pl_v, [jnp.full((16,), j, jnp.int32)])
            den2 = jnp.maximum(ln2.astype(jnp.float32), one)
            mlp_v[j, pl.ds(ED + TD, ED)] = acc2 / den2

        pltpu.sync_copy(mlp_v, out_hbm.at[pl.ds(b0, BW)])

    return k(text_table, event_table, text_idx, prev_idx, e1, tlen, plen)


def _matmul_body(wt_ref, mlp_ref, b_ref, out_ref):
    acc = lax.dot_general(
        wt_ref[...], mlp_ref[...], (((0,), (1,)), ((), ())),
        preferred_element_type=jnp.float32, precision=lax.Precision.HIGHEST)
    out_ref[...] = acc + b_ref[...].T


BN = 2048


def kernel(e1, e1_text_tokens, e1_text_lengths, e1prev_tokens, e1prev_lengths,
           event_table, text_table, W, b):
    EV = W.shape[0]
    text_idx = e1_text_tokens.reshape(-1).astype(jnp.int32)
    prev_idx = e1prev_tokens.reshape(-1).astype(jnp.int32)

    mlp = _sc_gather_pool(
        text_table, event_table, text_idx, prev_idx,
        e1.astype(jnp.int32),
        e1_text_lengths.astype(jnp.int32),
        e1prev_lengths.astype(jnp.int32))

    nblk = (EV + BN - 1) // BN
    logits_t = pl.pallas_call(
        _matmul_body,
        grid=(nblk,),
        in_specs=[
            pl.BlockSpec((FD, BN), lambda i: (0, i)),
            pl.BlockSpec((B, FD), lambda i: (0, 0)),
            pl.BlockSpec((1, BN), lambda i: (0, i)),
        ],
        out_specs=pl.BlockSpec((BN, B), lambda i: (i, 0)),
        out_shape=jax.ShapeDtypeStruct((EV, B), jnp.float32),
        compiler_params=pltpu.CompilerParams(
            dimension_semantics=("arbitrary",)),
    )(W.T, mlp, b.reshape(1, EV))
    return logits_t.T


# speedup vs baseline: 2.3061x; 1.5807x over previous
import dataclasses
import functools

import jax
import jax.numpy as jnp
from jax import lax
from jax.experimental import pallas as pl
from jax.experimental.pallas import tpu as pltpu
from jax.experimental.pallas import tpu_sc as plsc

B = 1024
T = 50
P = 20
ED = 16
TD = 16
FD = ED + TD + ED

NC = 2
NS = 16
NW = NC * NS
BW = B // NW
TEXT_PER_W = BW * T
PREV_PER_W = BW * P
GCHUNK = 128
PAD_ROW = 1


def _chunks(total):
    offs = []
    o = 0
    while o < total:
        offs.append((o, min(GCHUNK, total - o)))
        o += GCHUNK
    return offs


def _iota16():
    return lax.broadcasted_iota(jnp.int32, (16,), 0)


def _mask_indices(idx_v, len_v, n, stride):
    iota = _iota16()
    pad = jnp.full((16,), PAD_ROW, jnp.int32)
    stride_v = jnp.full((16,), stride, jnp.int32)

    @pl.loop(0, n // 16)
    def _(k):
        p = iota + k * 16
        belem = lax.div(p, stride_v)
        pos = p - belem * stride_v
        lens = plsc.load_gather(len_v, [belem])
        idx16 = idx_v[pl.ds(k * 16, 16)]
        idx_v[pl.ds(k * 16, 16)] = jnp.where(pos < lens, idx16, pad)


def _sc_compiler_params():
    cp = pltpu.CompilerParams(use_tc_tiling_on_sc=False)
    if "needs_layout_passes" in pltpu.CompilerParams.__dataclass_fields__:
        cp = dataclasses.replace(cp, needs_layout_passes=False)
    return cp


def _sc_gather_pool(text_table, event_table, text_idx, prev_idx, e1,
                    tlen, plen):
    mesh = plsc.VectorSubcoreMesh(core_axis_name="c", subcore_axis_name="s")

    @functools.partial(
        pl.kernel,
        mesh=mesh,
        compiler_params=_sc_compiler_params(),
        out_type=jax.ShapeDtypeStruct((B, FD), jnp.float32),
        scratch_types=[
            pltpu.VMEM((TEXT_PER_W,), jnp.int32),
            pltpu.VMEM((PREV_PER_W,), jnp.int32),
            pltpu.VMEM((BW,), jnp.int32),
            pltpu.VMEM((BW,), jnp.int32),
            pltpu.VMEM((BW,), jnp.int32),
            pltpu.VMEM((TEXT_PER_W, TD), jnp.float32),
            pltpu.VMEM((PREV_PER_W, ED), jnp.float32),
            pltpu.VMEM((BW, ED), jnp.float32),
            pltpu.VMEM((BW, FD), jnp.float32),
            pltpu.SemaphoreType.DMA,
        ],
    )
    def k(tt_hbm, et_hbm, ti_hbm, pi_hbm, e1_hbm, tl_hbm, pl_hbm, out_hbm,
          ti_v, pi_v, e1_v, tl_v, pl_v, rt_v, rp_v, re_v, mlp_v, sem):
        wid = lax.axis_index("s") * NC + lax.axis_index("c")
        b0 = wid * BW
        pltpu.sync_copy(ti_hbm.at[pl.ds(b0 * T, TEXT_PER_W)], ti_v)
        pltpu.sync_copy(pi_hbm.at[pl.ds(b0 * P, PREV_PER_W)], pi_v)
        pltpu.sync_copy(e1_hbm.at[pl.ds(b0, BW)], e1_v)
        pltpu.sync_copy(tl_hbm.at[pl.ds(b0, BW)], tl_v)
        pltpu.sync_copy(pl_hbm.at[pl.ds(b0, BW)], pl_v)

        _mask_indices(ti_v, tl_v, TEXT_PER_W, T)
        _mask_indices(pi_v, pl_v, PREV_PER_W, P)

        handles = []
        for off, sz in _chunks(TEXT_PER_W):
            handles.append(pltpu.async_copy(
                tt_hbm.at[ti_v.at[pl.ds(off, sz)]],
                rt_v.at[pl.ds(off, sz)], sem))
        for off, sz in _chunks(PREV_PER_W):
            handles.append(pltpu.async_copy(
                et_hbm.at[pi_v.at[pl.ds(off, sz)]],
                rp_v.at[pl.ds(off, sz)], sem))
        handles.append(pltpu.async_copy(et_hbm.at[e1_v], re_v, sem))
        for h in handles:
            h.wait()

        one = jnp.full((16,), 1.0, jnp.float32)

        @pl.loop(0, BW)
        def _(j):
            mlp_v[j, pl.ds(0, ED)] = re_v[j, :]

            acc = jnp.zeros((TD,), jnp.float32)
            tb = j * T
            for t in range(T):
                acc = acc + rt_v[tb + t, :]
            ln = plsc.load_gather(tl_v, [jnp.full((16,), j, jnp.int32)])
            den = jnp.maximum(ln.astype(jnp.float32), one)
            mlp_v[j, pl.ds(ED, TD)] = acc / den

            acc2 = jnp.zeros((ED,), jnp.float32)
            pb = j * P
            for t in range(P):
                acc2 = acc2 + rp_v[pb + t, :]
            ln2 = plsc.load_gather(pl_v, [jnp.full((16,), j, jnp.int32)])
            den2 = jnp.maximum(ln2.astype(jnp.float32), one)
            mlp_v[j, pl.ds(ED + TD, ED)] = acc2 / den2

        pltpu.sync_copy(mlp_v, out_hbm.at[pl.ds(b0, BW)])

    return k(text_table, event_table, text_idx, prev_idx, e1, tlen, plen)


def _matmul_body(wt_ref, mlp_ref, b_ref, out_ref):
    acc = lax.dot_general(
        wt_ref[...].astype(jnp.bfloat16), mlp_ref[...].astype(jnp.bfloat16),
        (((0,), (1,)), ((), ())),
        preferred_element_type=jnp.float32)
    out_ref[...] = acc + b_ref[...].T


BN = 2048


def kernel(e1, e1_text_tokens, e1_text_lengths, e1prev_tokens, e1prev_lengths,
           event_table, text_table, W, b):
    EV = W.shape[0]
    text_idx = e1_text_tokens.reshape(-1).astype(jnp.int32)
    prev_idx = e1prev_tokens.reshape(-1).astype(jnp.int32)

    mlp = _sc_gather_pool(
        text_table, event_table, text_idx, prev_idx,
        e1.astype(jnp.int32),
        e1_text_lengths.astype(jnp.int32),
        e1prev_lengths.astype(jnp.int32))

    nblk = (EV + BN - 1) // BN
    logits_t = pl.pallas_call(
        _matmul_body,
        grid=(nblk,),
        in_specs=[
            pl.BlockSpec((FD, BN), lambda i: (0, i)),
            pl.BlockSpec((B, FD), lambda i: (0, 0)),
            pl.BlockSpec((1, BN), lambda i: (0, i)),
        ],
        out_specs=pl.BlockSpec((BN, B), lambda i: (i, 0)),
        out_shape=jax.ShapeDtypeStruct((EV, B), jnp.float32),
        compiler_params=pltpu.CompilerParams(
            dimension_semantics=("arbitrary",)),
    )(W.T, mlp, b.reshape(1, EV))
    return logits_t.T
